# Initial kernel scaffold; baseline (speedup 1.0000x reference)
import jax
import jax.numpy as jnp
from jax.experimental import pallas as pl


def kernel(img_idx, poses, dR_param, dT_param):
    raise NotImplementedError("write your pallas kernel here")


# speedup vs baseline: 1.8066x; 1.8066x over previous
import functools

import jax
import jax.numpy as jnp
from jax import lax
from jax.experimental import pallas as pl
from jax.experimental.pallas import tpu as pltpu
from jax.experimental.pallas import tpu_sc as plsc

_B = 16384
_NC, _NS, _L = 2, 16, 16
_NW = _NC * _NS
_CHUNK = _B // _NW
_STEPS = _CHUNK // _L
_GCH = 128
_NG = _CHUNK // _GCH
_NI = 3 * _NG

_mesh = plsc.VectorSubcoreMesh(core_axis_name="c", subcore_axis_name="s")


@functools.partial(
    pl.kernel,
    mesh=_mesh,
    out_type=jax.ShapeDtypeStruct((_NW, 12 * _CHUNK), jnp.float32),
    scratch_types=[
        pltpu.VMEM((_NG, _GCH), jnp.int32),
        pltpu.VMEM((_NI, _GCH), jnp.int32),
        pltpu.VMEM((3 * _CHUNK,), jnp.float32),
        pltpu.VMEM((3 * _CHUNK,), jnp.float32),
        pltpu.VMEM((12 * _CHUNK,), jnp.float32),
        pltpu.VMEM((12 * _CHUNK,), jnp.float32),
        pltpu.SemaphoreType.DMA,
        pltpu.SemaphoreType.DMA,
    ],
)
def _extr_pose_sc(idx_hbm, poses_hbm, dR_hbm, dT_hbm, out_hbm,
                  idx_v, idx3_v, dRv, dTv, pv, ov, gsem, psem):
    wid = lax.axis_index("s") * _NC + lax.axis_index("c")

    pltpu.sync_copy(idx_hbm.at[wid], idx_v)
    pcopy = pltpu.async_copy(poses_hbm.at[wid], pv, psem)

    for s in range(_STEPS):
        j, o = s // (_GCH // _L), (s % (_GCH // _L)) * _L
        x3 = idx_v[j, pl.ds(o, _L)] * 3
        for c in range(3):
            idx3_v[c * _NG + j, pl.ds(o, _L)] = x3 + c

    gcopies = []
    for r in range(_NI):
        dst = pl.ds(r * _GCH, _GCH)
        gcopies.append(pltpu.async_copy(dR_hbm.at[idx3_v.at[r]],
                                        dRv.at[dst], gsem))
        gcopies.append(pltpu.async_copy(dT_hbm.at[idx3_v.at[r]],
                                        dTv.at[dst], gsem))
    for copy in gcopies:
        copy.wait()
    pcopy.wait()

    def body(s, carry):
        off = s * _L

        def ld(ref, c):
            return ref[pl.ds(c * _CHUNK + off, _L)]

        v0, v1, v2 = ld(dRv, 0), ld(dRv, 1), ld(dRv, 2)
        p = [ld(pv, c) for c in range(12)]

        n2 = v0 * v0 + v1 * v1 + v2 * v2
        a = 1.0 + n2 * (-1.0 / 6.0 + n2 * (1.0 / 120.0 + n2 * (
            -1.0 / 5040.0 + n2 * (1.0 / 362880.0))))
        b = 0.5 + n2 * (-1.0 / 24.0 + n2 * (1.0 / 720.0 + n2 * (
            -1.0 / 40320.0 + n2 * (1.0 / 3628800.0))))
        c1 = 1.0 - b * n2
        bv0, bv1, bv2 = b * v0, b * v1, b * v2
        av0, av1, av2 = a * v0, a * v1, a * v2
        R = [
            c1 + bv0 * v0, bv0 * v1 - av2, bv0 * v2 + av1,
            bv0 * v1 + av2, c1 + bv1 * v1, bv1 * v2 - av0,
            bv0 * v2 - av1, bv1 * v2 + av0, c1 + bv2 * v2,
        ]
        for i in range(3):
            for j in range(3):
                val = (R[3 * i] * p[j] + R[3 * i + 1] * p[4 + j]
                       + R[3 * i + 2] * p[8 + j])
                ov[pl.ds((4 * i + j) * _CHUNK + off, _L)] = val
        for i in range(3):
            val = p[4 * i + 3] + ld(dTv, i)
            ov[pl.ds((4 * i + 3) * _CHUNK + off, _L)] = val
        return carry

    lax.fori_loop(0, _STEPS, body, 0)
    pltpu.sync_copy(ov, out_hbm.at[wid])


def kernel(img_idx, poses, dR_param, dT_param):
    idx3d = img_idx.reshape(_NW, _NG, _GCH)
    posesw = poses.reshape(_NW, _CHUNK, 12).transpose(0, 2, 1)
    posesw = posesw.reshape(_NW, 12 * _CHUNK)
    out = _extr_pose_sc(idx3d, posesw, dR_param.reshape(-1),
                        dT_param.reshape(-1))
    out = out.reshape(_NW, 12, _CHUNK).transpose(0, 2, 1)
    return out.reshape(_B, 3, 4)
